# baseline (speedup 1.0000x reference)
import functools

import jax
import jax.numpy as jnp
import numpy as np
from jax.experimental import pallas as pl
from jax.experimental.pallas import tpu as pltpu


def _fused_kernel(eall_ref, x_ref, gf_ref, uf_ref, gbp_ref, ubp_ref,
                  lw_ref, lb_ref, pmat_ref, ssel_ref, posg_ref, posu_ref,
                  pb_ref, g1_ref, g2_ref, t2_ref, sum2_ref, out_ref,
                  *, n_rows, ncols, cin, emb_dim, chunk):
    cw = gf_ref.shape[0] // (2 * emb_dim)
    w_k = []
    for k in range(2):
        acc = None
        for d in range(emb_dim):
            off = (2 * d + k) * cw
            term = (jnp.dot(gf_ref[off:off + cin, :], posg_ref[d],
                            preferred_element_type=jnp.float32)
                    + jnp.dot(uf_ref[off:off + cin, :], posu_ref[d],
                              preferred_element_type=jnp.float32))
            acc = term if acc is None else acc + term
        w_k.append(acc)
    nb = x_ref.shape[0]
    u = None
    v = None
    for b in range(nb):
        wab = jnp.dot(w_k[0], pb_ref[b], preferred_element_type=jnp.float32)
        wbb = jnp.dot(w_k[1], pb_ref[b], preferred_element_type=jnp.float32)
        du = jnp.dot(x_ref[b], wab, preferred_element_type=jnp.float32)
        dv = jnp.dot(x_ref[b], wbb, preferred_element_type=jnp.float32)
        u = du if u is None else u + du
        v = dv if v is None else v + dv
    va = jnp.concatenate([v, jnp.ones((n_rows, 1), jnp.float32)], axis=1)

    ea = eall_ref[...]
    eas = ea * np.float32(1.4426950408889634)
    pv = None
    for c0 in range(0, n_rows, chunk):
        w = min(chunk, n_rows - c0)
        ec = eall_ref[c0:c0 + w, :]
        a = jax.lax.dot_general(eas, ec, (((1,), (1,)), ((), ())),
                                preferred_element_type=jnp.float32)
        p = jnp.exp2(jnp.minimum(jnp.maximum(a, 0.0), 122.0))
        term = jnp.dot(p, va[c0:c0 + w, :], preferred_element_type=jnp.float32)
        pv = term if pv is None else pv + term
    inv = 1.0 / pv[:, ncols:ncols + 1]

    t = u + pv[:, 0:ncols] * inv
    emul = jnp.dot(ea, pmat_ref[...], preferred_element_type=jnp.float32)
    gu = (jnp.dot(t * emul, ssel_ref[...], preferred_element_type=jnp.float32)
          + jnp.dot(jnp.dot(ea, gbp_ref[...],
                            preferred_element_type=jnp.float32),
                    g1_ref[...], preferred_element_type=jnp.float32)
          + jnp.dot(jnp.dot(ea, ubp_ref[...],
                            preferred_element_type=jnp.float32),
                    g2_ref[...], preferred_element_type=jnp.float32))
    r = jax.nn.sigmoid(gu[:, 0:8])
    h = jnp.tanh(gu[:, 8:16])
    y = jnp.maximum((1.0 - r) * h, 0.0)
    lwt = jnp.dot(lw_ref[...], t2_ref[...],
                  preferred_element_type=jnp.float32)
    yo = (jnp.dot(y * lwt, sum2_ref[...], preferred_element_type=jnp.float32)
          + lb_ref[0:1, 0:1])
    out_ref[...] = yo.T


def kernel(x, e, gate_weights_pool, gate_bias_pool, update_weights_pool,
           update_bias_pool, linear_w, linear_b):
    B, N, C = x.shape
    D = e.shape[1]
    Cw = gate_weights_pool.shape[2]
    nc = 24 * B

    gflat = gate_weights_pool.reshape(D * 2 * Cw, 4)
    uflat = update_weights_pool.reshape(D * 2 * Cw, 2)
    lb2 = linear_b.reshape(1, 1)

    posg = np.zeros((D, 4, 24), np.float32)
    posu = np.zeros((D, 2, 24), np.float32)
    for d in range(D):
        for o in range(4):
            posg[d, o, 4 * d + o] = 1.0
        for o in range(2):
            posu[d, o, 16 + 2 * d + o] = 1.0
    pbmat = np.zeros((B, 24, nc), np.float32)
    for b in range(B):
        for c in range(24):
            pbmat[b, c, 24 * b + c] = 1.0
    pm1 = np.zeros((D, 24), np.float32)
    for d in range(D):
        pm1[d, 4 * d:4 * d + 4] = 1.0
        pm1[d, 16 + 2 * d:16 + 2 * d + 2] = 1.0
    pmat = np.tile(pm1, (1, B))
    ss1 = np.zeros((24, 16), np.float32)
    for d in range(D):
        for j in range(2):
            ss1[4 * d + 2 + j, j] = 1.0
            ss1[16 + 2 * d + j, 8 + j] = 1.0
    ssel = np.zeros((nc, 16), np.float32)
    for b in range(B):
        ssel[b * 24:(b + 1) * 24, 2 * b:2 * b + 2] = ss1[:, 0:2]
        ssel[b * 24:(b + 1) * 24, 8 + 2 * b:8 + 2 * b + 2] = ss1[:, 8:10]
    g1 = np.zeros((4, 16), np.float32)
    g2 = np.zeros((2, 16), np.float32)
    for b in range(B):
        for j in range(2):
            g1[2 + j, 2 * b + j] = 1.0
            g2[j, 8 + 2 * b + j] = 1.0
    t2 = np.zeros((2, 2 * B), np.float32)
    for b in range(B):
        for j in range(2):
            t2[j, 2 * b + j] = 1.0
    sum2 = np.zeros((2 * B, B), np.float32)
    for b in range(B):
        for j in range(2):
            sum2[2 * b + j, b] = 1.0

    consts = [jnp.asarray(arr) for arr in
              (pmat, ssel, posg, posu, pbmat, g1, g2, t2, sum2)]

    full = lambda *shape: pl.BlockSpec(shape, lambda: tuple(0 for _ in shape))
    y2 = pl.pallas_call(
        functools.partial(_fused_kernel, n_rows=N, ncols=nc, cin=C,
                          emb_dim=D, chunk=512),
        in_specs=[
            full(N, D),
            full(B, N, C),
            full(D * 2 * Cw, 4),
            full(D * 2 * Cw, 2),
            full(D, 4),
            full(D, 2),
            full(1, 2),
            full(1, 1),
            full(D, nc),
            full(nc, 4 * B),
            full(D, 4, 24),
            full(D, 2, 24),
            full(B, 24, nc),
            full(4, 4 * B),
            full(2, 4 * B),
            full(2, 2 * B),
            full(2 * B, B),
        ],
        out_specs=full(B, N),
        out_shape=jax.ShapeDtypeStruct((B, N), jnp.float32),
        compiler_params=pltpu.CompilerParams(
            dimension_semantics=(),
        ),
    )(e, x, gflat, uflat, gate_bias_pool, update_bias_pool,
      linear_w, lb2, *consts)

    return y2[:, :, None]
